# baseline (speedup 1.0000x reference)
import jax
import jax.numpy as jnp
from jax.experimental import pallas as pl
from jax.experimental.pallas import tpu as pltpu

_B = 16
_TOL = 0.5
_S = 4
_ROWS = 32
_SLOTS = 4


def _flags_kernel(m_ref, fl_ref, gf_ref):
    hh, ww = m_ref.shape[1], m_ref.shape[2]
    nbh, nbw = hh // _B, ww // _B
    m = m_ref[0]
    ri = jax.lax.broadcasted_iota(jnp.int32, (nbh, hh), 0)
    rj = jax.lax.broadcasted_iota(jnp.int32, (nbh, hh), 1) // _B
    p = (ri == rj).astype(jnp.float32)
    ci = jax.lax.broadcasted_iota(jnp.int32, (ww, nbw), 0) // _B
    cj = jax.lax.broadcasted_iota(jnp.int32, (ww, nbw), 1)
    s = (ci == cj).astype(jnp.float32)
    hi = jax.lax.Precision.HIGHEST
    rowsum = jnp.dot(p, m, precision=hi,
                     preferred_element_type=jnp.float32)
    blocksum = jnp.dot(rowsum, s, precision=hi,
                       preferred_element_type=jnp.float32)
    active = blocksum > (_TOL * _B * _B)
    fl_ref[0] = active.astype(jnp.int32)
    gf_ref[0] = jnp.dot(active.astype(jnp.float32), s.T,
                        preferred_element_type=jnp.float32)


def _make_main(n, h, w, c, f):
    nj = h // _ROWS
    total = n * nj
    nb = _ROWS // _B
    nbw = w // _B
    chunk = w // _S

    def body(x_ref, bl_ref, cnt_ref, gf_ref, w_ref, b_ref, o_ref,
             xscr, oscr, sin, sout):
        i = pl.program_id(0)
        j = pl.program_id(1)
        t = i * nj + j
        slot = jax.lax.rem(t, 3)
        oslot = jax.lax.rem(t, _SLOTS)

        def in_block_copy(step_i, step_j, br, bw, xslot, sidx):
            return pltpu.make_async_copy(
                x_ref.at[step_i,
                         pl.ds(step_j * _ROWS + br * _B, _B),
                         pl.ds(bw * _B, _B), :],
                xscr.at[xslot, pl.ds(br * _B, _B), pl.ds(bw * _B, _B), :],
                sin.at[xslot, sidx])

        def issue_in(step, xslot):
            si = step // nj
            sj = jax.lax.rem(step, nj)

            def do(idx, _):
                bid = bl_ref[si, sj, idx]
                br = bid // nbw
                bw = jax.lax.rem(bid, nbw)
                in_block_copy(si, sj, br, bw, xslot, idx).start()
                return 0

            jax.lax.fori_loop(0, cnt_ref[si, sj], do, 0)

        def wait_in(step, xslot):
            si = step // nj
            sj = jax.lax.rem(step, nj)

            def do(idx, _):
                bid = bl_ref[si, sj, idx]
                br = bid // nbw
                bw = jax.lax.rem(bid, nbw)
                in_block_copy(si, sj, br, bw, xslot, idx).wait()
                return 0

            jax.lax.fori_loop(0, cnt_ref[si, sj], do, 0)

        def out_dst(step_i, step_j, k):
            return o_ref.at[step_i, pl.ds(step_j * _ROWS, _ROWS),
                            pl.ds(k * chunk, chunk), :]

        @pl.when(t == 0)
        def _():
            issue_in(0, 0)
            issue_in(1, 1)

        wait_in(t, slot)

        @pl.when(t + 2 < total)
        def _():
            issue_in(t + 2, jax.lax.rem(t + 2, 3))

        @pl.when(t >= _SLOTS)
        def _():
            ti = (t - _SLOTS) // nj
            tj = jax.lax.rem(t - _SLOTS, nj)
            for k in range(_S):
                pltpu.make_async_copy(
                    oscr.at[oslot, :, pl.ds(k * chunk, chunk), :],
                    out_dst(ti, tj, k),
                    sout.at[oslot, k]).wait()

        gate = gf_ref[0, 0]
        gate_t = gate.T

        for k in range(_S):
            x = xscr[slot, :, k * chunk:(k + 1) * chunk, :]
            y = jax.lax.dot_general(
                x.reshape(_ROWS * chunk, c), w_ref[...],
                (((1,), (0,)), ((), ())),
                preferred_element_type=jnp.float32,
            ) + b_ref[...]
            y = y.reshape(nb, _B, chunk, f)
            g = gate_t[k * chunk:(k + 1) * chunk]
            for br in range(nb):
                oscr[oslot, br * _B:(br + 1) * _B,
                     k * chunk:(k + 1) * chunk, :] = jnp.where(
                         g[:, br][None, :, None] > 0.5, y[br], 0.0)

        for k in range(_S):
            pltpu.make_async_copy(
                oscr.at[oslot, :, pl.ds(k * chunk, chunk), :],
                out_dst(i, j, k),
                sout.at[oslot, k]).start()

        @pl.when(t == total - 1)
        def _():
            for tt in range(max(total - _SLOTS, 0), total):
                ti, tj = divmod(tt, nj)
                for k in range(_S):
                    pltpu.make_async_copy(
                        oscr.at[tt % _SLOTS, :, pl.ds(k * chunk, chunk), :],
                        out_dst(ti, tj, k),
                        sout.at[tt % _SLOTS, k]).wait()

    return body


def kernel(inputs, mask, weights, bias):
    n, h, w, c = inputs.shape
    f = weights.shape[-1]
    nbh, nbw = h // _B, w // _B
    chunk = w // _S

    w2 = weights.reshape(c, f)
    b2 = bias.reshape(1, f)
    m2 = mask.reshape(n, h, w)

    flags, gatef = pl.pallas_call(
        _flags_kernel,
        grid=(n,),
        in_specs=[pl.BlockSpec((1, h, w), lambda i: (i, 0, 0))],
        out_specs=[
            pl.BlockSpec((1, nbh, nbw), lambda i: (i, 0, 0)),
            pl.BlockSpec((1, nbh, w), lambda i: (i, 0, 0)),
        ],
        out_shape=[
            jax.ShapeDtypeStruct((n, nbh, nbw), jnp.int32),
            jax.ShapeDtypeStruct((n, nbh, w), jnp.float32),
        ],
    )(m2)

    nj = h // _ROWS
    nb = _ROWS // _B
    strip_flags = flags.reshape(n, nj, nb * nbw)
    order = jnp.broadcast_to(jnp.arange(nb * nbw, dtype=jnp.int32),
                             strip_flags.shape)
    counts = jnp.full(strip_flags.shape[:2], nb * nbw, dtype=jnp.int32)
    blist = order

    out = pl.pallas_call(
        _make_main(n, h, w, c, f),
        grid=(n, nj),
        in_specs=[
            pl.BlockSpec(memory_space=pltpu.MemorySpace.HBM),
            pl.BlockSpec(memory_space=pltpu.MemorySpace.SMEM),
            pl.BlockSpec(memory_space=pltpu.MemorySpace.SMEM),
            pl.BlockSpec((1, 1, nb, w), lambda i, j: (i, j, 0, 0)),
            pl.BlockSpec((c, f), lambda i, j: (0, 0)),
            pl.BlockSpec((1, f), lambda i, j: (0, 0)),
        ],
        out_specs=pl.BlockSpec(memory_space=pltpu.MemorySpace.HBM),
        out_shape=jax.ShapeDtypeStruct((n, h, w, f), jnp.float32),
        scratch_shapes=[
            pltpu.VMEM((3, _ROWS, w, c), jnp.float32),
            pltpu.VMEM((_SLOTS, _ROWS, w, f), jnp.float32),
            pltpu.SemaphoreType.DMA((3, nb * nbw)),
            pltpu.SemaphoreType.DMA((_SLOTS, _S)),
        ],
        compiler_params=pltpu.CompilerParams(
            dimension_semantics=("arbitrary", "arbitrary")),
    )(inputs, blist, counts, gatef.reshape(n, nj, nb, w), w2, b2)
    return out


# speedup vs baseline: 1.0605x; 1.0605x over previous
import jax
import jax.numpy as jnp
from jax.experimental import pallas as pl
from jax.experimental.pallas import tpu as pltpu

_B = 16
_TOL = 0.5
_S = 4
_ROWS = 32
_SLOTS = 4


def _flags_kernel(m_ref, fl_ref, gf_ref):
    hh, ww = m_ref.shape[1], m_ref.shape[2]
    nbh, nbw = hh // _B, ww // _B
    m = m_ref[0]
    ri = jax.lax.broadcasted_iota(jnp.int32, (nbh, hh), 0)
    rj = jax.lax.broadcasted_iota(jnp.int32, (nbh, hh), 1) // _B
    p = (ri == rj).astype(jnp.float32)
    ci = jax.lax.broadcasted_iota(jnp.int32, (ww, nbw), 0) // _B
    cj = jax.lax.broadcasted_iota(jnp.int32, (ww, nbw), 1)
    s = (ci == cj).astype(jnp.float32)
    hi = jax.lax.Precision.HIGHEST
    rowsum = jnp.dot(p, m, precision=hi,
                     preferred_element_type=jnp.float32)
    blocksum = jnp.dot(rowsum, s, precision=hi,
                       preferred_element_type=jnp.float32)
    active = blocksum > (_TOL * _B * _B)
    fl_ref[0] = active.astype(jnp.int32)
    gf_ref[0] = jnp.dot(active.astype(jnp.float32), s.T,
                        preferred_element_type=jnp.float32)


def _make_main(n, h, w, c, f):
    nj = h // _ROWS
    total = n * nj
    nb = _ROWS // _B
    nbw = w // _B
    chunk = w // _S

    def body(x_ref, bl_ref, cnt_ref, gf_ref, w_ref, b_ref, o_ref,
             xscr, oscr, sin, sout):
        i = pl.program_id(0)
        j = pl.program_id(1)
        t = i * nj + j
        slot = jax.lax.rem(t, 3)
        oslot = jax.lax.rem(t, _SLOTS)

        def in_block_copy(step_i, step_j, br, bw, xslot, sidx):
            return pltpu.make_async_copy(
                x_ref.at[step_i,
                         pl.ds(step_j * _ROWS + br * _B, _B),
                         pl.ds(bw * _B, _B), :],
                xscr.at[xslot, pl.ds(br * _B, _B), pl.ds(bw * _B, _B), :],
                sin.at[xslot, sidx])

        def issue_in(step, xslot):
            si = step // nj
            sj = jax.lax.rem(step, nj)

            def do(idx, _):
                bid = bl_ref[si, sj, idx]
                br = bid // nbw
                bw = jax.lax.rem(bid, nbw)
                in_block_copy(si, sj, br, bw, xslot, idx).start()
                return 0

            jax.lax.fori_loop(0, cnt_ref[si, sj], do, 0)

        def wait_in(step, xslot):
            si = step // nj
            sj = jax.lax.rem(step, nj)

            def do(idx, _):
                bid = bl_ref[si, sj, idx]
                br = bid // nbw
                bw = jax.lax.rem(bid, nbw)
                in_block_copy(si, sj, br, bw, xslot, idx).wait()
                return 0

            jax.lax.fori_loop(0, cnt_ref[si, sj], do, 0)

        def out_dst(step_i, step_j, k):
            return o_ref.at[step_i, pl.ds(step_j * _ROWS, _ROWS),
                            pl.ds(k * chunk, chunk), :]

        @pl.when(t == 0)
        def _():
            issue_in(0, 0)
            issue_in(1, 1)

        wait_in(t, slot)

        @pl.when(t + 2 < total)
        def _():
            issue_in(t + 2, jax.lax.rem(t + 2, 3))

        @pl.when(t >= _SLOTS)
        def _():
            ti = (t - _SLOTS) // nj
            tj = jax.lax.rem(t - _SLOTS, nj)
            for k in range(_S):
                pltpu.make_async_copy(
                    oscr.at[oslot, :, pl.ds(k * chunk, chunk), :],
                    out_dst(ti, tj, k),
                    sout.at[oslot, k]).wait()

        gate = gf_ref[0, 0]
        gate_t = gate.T

        for k in range(_S):
            x = xscr[slot, :, k * chunk:(k + 1) * chunk, :]
            y = jax.lax.dot_general(
                x.reshape(_ROWS * chunk, c), w_ref[...],
                (((1,), (0,)), ((), ())),
                preferred_element_type=jnp.float32,
            ) + b_ref[...]
            y = y.reshape(nb, _B, chunk, f)
            g = gate_t[k * chunk:(k + 1) * chunk]
            for br in range(nb):
                oscr[oslot, br * _B:(br + 1) * _B,
                     k * chunk:(k + 1) * chunk, :] = jnp.where(
                         g[:, br][None, :, None] > 0.5, y[br], 0.0)

        for k in range(_S):
            pltpu.make_async_copy(
                oscr.at[oslot, :, pl.ds(k * chunk, chunk), :],
                out_dst(i, j, k),
                sout.at[oslot, k]).start()

        @pl.when(t == total - 1)
        def _():
            for tt in range(max(total - _SLOTS, 0), total):
                ti, tj = divmod(tt, nj)
                for k in range(_S):
                    pltpu.make_async_copy(
                        oscr.at[tt % _SLOTS, :, pl.ds(k * chunk, chunk), :],
                        out_dst(ti, tj, k),
                        sout.at[tt % _SLOTS, k]).wait()

    return body


def kernel(inputs, mask, weights, bias):
    n, h, w, c = inputs.shape
    f = weights.shape[-1]
    nbh, nbw = h // _B, w // _B
    chunk = w // _S

    w2 = weights.reshape(c, f)
    b2 = bias.reshape(1, f)
    m2 = mask.reshape(n, h, w)

    flags, gatef = pl.pallas_call(
        _flags_kernel,
        grid=(n,),
        in_specs=[pl.BlockSpec((1, h, w), lambda i: (i, 0, 0))],
        out_specs=[
            pl.BlockSpec((1, nbh, nbw), lambda i: (i, 0, 0)),
            pl.BlockSpec((1, nbh, w), lambda i: (i, 0, 0)),
        ],
        out_shape=[
            jax.ShapeDtypeStruct((n, nbh, nbw), jnp.int32),
            jax.ShapeDtypeStruct((n, nbh, w), jnp.float32),
        ],
    )(m2)

    nj = h // _ROWS
    nb = _ROWS // _B
    strip_flags = flags.reshape(n, nj, nb * nbw)
    order = jnp.argsort(1 - strip_flags, axis=-1, stable=True)
    counts = jnp.sum(strip_flags, axis=-1, dtype=jnp.int32)
    blist = order.astype(jnp.int32)

    out = pl.pallas_call(
        _make_main(n, h, w, c, f),
        grid=(n, nj),
        in_specs=[
            pl.BlockSpec(memory_space=pltpu.MemorySpace.HBM),
            pl.BlockSpec(memory_space=pltpu.MemorySpace.SMEM),
            pl.BlockSpec(memory_space=pltpu.MemorySpace.SMEM),
            pl.BlockSpec((1, 1, nb, w), lambda i, j: (i, j, 0, 0)),
            pl.BlockSpec((c, f), lambda i, j: (0, 0)),
            pl.BlockSpec((1, f), lambda i, j: (0, 0)),
        ],
        out_specs=pl.BlockSpec(memory_space=pltpu.MemorySpace.HBM),
        out_shape=jax.ShapeDtypeStruct((n, h, w, f), jnp.float32),
        scratch_shapes=[
            pltpu.VMEM((3, _ROWS, w, c), jnp.float32),
            pltpu.VMEM((_SLOTS, _ROWS, w, f), jnp.float32),
            pltpu.SemaphoreType.DMA((3, nb * nbw)),
            pltpu.SemaphoreType.DMA((_SLOTS, _S)),
        ],
        compiler_params=pltpu.CompilerParams(
            dimension_semantics=("arbitrary", "arbitrary")),
    )(inputs, blist, counts, gatef.reshape(n, nj, nb, w), w2, b2)
    return out
